# baseline (speedup 1.0000x reference)
import functools

import jax
import jax.numpy as jnp
from jax import lax
from jax.experimental import pallas as pl
from jax.experimental.pallas import tpu as pltpu
from jax.experimental.pallas import tpu_sc as plsc

_DIM_G = 16
_DIM_S = 32
_DIM_O = _DIM_G + _DIM_S
_PACK_S = 128 // _DIM_S
_NC = 2
_NS = 16
_NW = _NC * _NS
_GRP = 16
_CROWS = 128
_L = 16


@functools.lru_cache(maxsize=None)
def _build(B):
    b_per_w = B // _NW
    n_grp = b_per_w // _GRP
    n_chunks = b_per_w // _CROWS
    grp_per_chunk = _CROWS // _GRP
    mesh = plsc.VectorSubcoreMesh(
        core_axis_name="c", subcore_axis_name="s",
        num_cores=_NC, num_subcores=_NS,
    )

    @functools.partial(
        pl.kernel,
        mesh=mesh,
        out_type=jax.ShapeDtypeStruct((B, _DIM_O), jnp.float32),
        scratch_types=[
            pltpu.VMEM((b_per_w,), jnp.int32),
            pltpu.VMEM((b_per_w,), jnp.int32),
            pltpu.VMEM((b_per_w,), jnp.int32),
            pltpu.VMEM((2, _GRP, 8, _DIM_G), jnp.float32),
            pltpu.VMEM((2, _CROWS, 128), jnp.float32),
            pltpu.VMEM((_CROWS, _DIM_O), jnp.float32),
            pltpu.SemaphoreType.DMA,
            pltpu.SemaphoreType.DMA,
        ],
    )
    def emb_kernel(gidx_hbm, sidx_hbm, wg_hbm, ws_hbm, out_hbm,
                   gidx_v, sidx_v, sR_v, g_buf, s_rows, out_c,
                   sem_g, sem_s):
        wid = lax.axis_index("s") * _NC + lax.axis_index("c")
        base = wid * b_per_w
        pltpu.sync_copy(gidx_hbm.at[pl.ds(base, b_per_w)], gidx_v)
        pltpu.sync_copy(sidx_hbm.at[pl.ds(base, b_per_w)], sidx_v)

        for k in range(b_per_w // _L):
            sl = pl.ds(k * _L, _L)
            sR_v[sl] = sidx_v[sl] >> 2

        def fire_sys(c, pc):
            rows = pl.ds(c * _CROWS, _CROWS)
            pltpu.async_copy(ws_hbm.at[sR_v.at[rows]], s_rows.at[pc], sem_s)

        def fire_graph(i, p):
            gvec = gidx_v[pl.ds(i * _GRP, _GRP)] >> 3
            for l in range(_GRP):
                pltpu.async_copy(wg_hbm.at[gvec[l]], g_buf.at[p, l], sem_g)

        fire_sys(0, 0)
        fire_graph(0, 0)

        def step(i, _):
            p = lax.rem(i, 2)

            @pl.when(i + 1 < n_grp)
            def _():
                fire_graph(i + 1, 1 - p)

            pltpu.make_async_copy(wg_hbm.at[pl.ds(0, _GRP)],
                                  g_buf.at[p], sem_g).wait()

            crow = lax.rem(i, grp_per_chunk) * _GRP
            gsub = lax.rem(gidx_v[pl.ds(i * _GRP, _GRP)], 8)
            for l in range(_GRP):
                row = crow + l
                out_c[row, pl.ds(0, _DIM_G)] = g_buf[p, l, gsub[l], :]
            return ()

        for c in range(n_chunks):
            pc = c % 2
            if c + 1 < n_chunks:
                fire_sys(c + 1, 1 - pc)
            pltpu.make_async_copy(ws_hbm.at[pl.ds(0, _CROWS)],
                                  s_rows.at[pc], sem_s).wait()

            lax.fori_loop(c * grp_per_chunk, (c + 1) * grp_per_chunk,
                          step, (), unroll=False)

            for grp in range(grp_per_chunk):
                crow = grp * _L
                soff = (sidx_v[pl.ds(c * _CROWS + crow, _L)]
                        & (_PACK_S - 1)) * _DIM_S
                for l in range(_L):
                    row = crow + l
                    out_c[row, pl.ds(_DIM_G, _L)] = \
                        s_rows[pc, row, pl.ds(soff[l], _L)]
                    out_c[row, pl.ds(_DIM_G + _L, _L)] = \
                        s_rows[pc, row, pl.ds(soff[l] + _L, _L)]
            pltpu.sync_copy(out_c, out_hbm.at[pl.ds(base + c * _CROWS, _CROWS)])

    return emb_kernel


@jax.jit
def kernel(graph_type, system_id, W_graph_type, W_system_id):
    B = graph_type.shape[0]
    ng, dg = W_graph_type.shape
    ns, ds = W_system_id.shape
    return _build(B)(
        graph_type.reshape(B), system_id.reshape(B),
        W_graph_type.reshape(ng // 8, 8, dg),
        W_system_id.reshape(ns // _PACK_S, _PACK_S * ds),
    )


# speedup vs baseline: 1.0158x; 1.0158x over previous
import functools

import jax
import jax.numpy as jnp
from jax import lax
from jax.experimental import pallas as pl
from jax.experimental.pallas import tpu as pltpu
from jax.experimental.pallas import tpu_sc as plsc

_DIM_G = 16
_DIM_S = 32
_DIM_O = _DIM_G + _DIM_S
_NC = 2
_NS = 16
_NW = _NC * _NS
_GRP = 16
_CROWS = 128


@functools.lru_cache(maxsize=None)
def _build(B):
    b_per_w = B // _NW
    n_grp = b_per_w // _GRP
    n_chunks = b_per_w // _CROWS
    grp_per_chunk = _CROWS // _GRP
    mesh = plsc.VectorSubcoreMesh(
        core_axis_name="c", subcore_axis_name="s",
        num_cores=_NC, num_subcores=_NS,
    )

    @functools.partial(
        pl.kernel,
        mesh=mesh,
        out_type=jax.ShapeDtypeStruct((B, _DIM_O), jnp.float32),
        scratch_types=[
            pltpu.VMEM((b_per_w,), jnp.int32),
            pltpu.VMEM((b_per_w,), jnp.int32),
            pltpu.VMEM((2, _GRP, 8, _DIM_G), jnp.float32),
            pltpu.VMEM((2, _GRP, 8, _DIM_S), jnp.float32),
            pltpu.VMEM((_CROWS, _DIM_O), jnp.float32),
            pltpu.SemaphoreType.DMA,
        ],
    )
    def emb_kernel(gidx_hbm, sidx_hbm, wg_hbm, ws_hbm, out_hbm,
                   gidx_v, sidx_v, g_buf, s_buf, out_c, sem):
        wid = lax.axis_index("s") * _NC + lax.axis_index("c")
        base = wid * b_per_w
        pltpu.sync_copy(gidx_hbm.at[pl.ds(base, b_per_w)], gidx_v)
        pltpu.sync_copy(sidx_hbm.at[pl.ds(base, b_per_w)], sidx_v)

        def fire(i, p):
            gvec = gidx_v[pl.ds(i * _GRP, _GRP)] >> 3
            svec = sidx_v[pl.ds(i * _GRP, _GRP)] >> 3
            for l in range(_GRP):
                pltpu.async_copy(wg_hbm.at[gvec[l]], g_buf.at[p, l], sem)
                pltpu.async_copy(ws_hbm.at[svec[l]], s_buf.at[p, l], sem)

        fire(0, 0)

        def step(i, _):
            p = lax.rem(i, 2)

            @pl.when(i + 1 < n_grp)
            def _():
                fire(i + 1, 1 - p)

            pltpu.make_async_copy(wg_hbm.at[pl.ds(0, _GRP)],
                                  g_buf.at[p], sem).wait()
            pltpu.make_async_copy(ws_hbm.at[pl.ds(0, _GRP)],
                                  s_buf.at[p], sem).wait()

            crow = lax.rem(i, grp_per_chunk) * _GRP
            gsub = lax.rem(gidx_v[pl.ds(i * _GRP, _GRP)], 8)
            ssub = lax.rem(sidx_v[pl.ds(i * _GRP, _GRP)], 8)
            for l in range(_GRP):
                row = crow + l
                out_c[row, pl.ds(0, _DIM_G)] = g_buf[p, l, gsub[l], :]
                out_c[row, pl.ds(_DIM_G, 16)] = s_buf[p, l, ssub[l], pl.ds(0, 16)]
                out_c[row, pl.ds(_DIM_G + 16, 16)] = s_buf[p, l, ssub[l], pl.ds(16, 16)]
            return ()

        for c in range(n_chunks):
            lax.fori_loop(c * grp_per_chunk, (c + 1) * grp_per_chunk,
                          step, (), unroll=False)
            pltpu.sync_copy(out_c, out_hbm.at[pl.ds(base + c * _CROWS, _CROWS)])

    return emb_kernel


@jax.jit
def kernel(graph_type, system_id, W_graph_type, W_system_id):
    B = graph_type.shape[0]
    ng, dg = W_graph_type.shape
    ns, ds = W_system_id.shape
    return _build(B)(
        graph_type.reshape(B), system_id.reshape(B),
        W_graph_type.reshape(ng // 8, 8, dg),
        W_system_id.reshape(ns // 8, 8, ds),
    )
